# baseline (speedup 1.0000x reference)
import functools

import jax
import jax.numpy as jnp
from jax import lax
from jax.experimental import pallas as pl
from jax.experimental.pallas import tpu as pltpu
from jax.experimental.pallas import tpu_sc as plsc

N = 10000
E = 320000
D = 128
NC, NS, L = 2, 16, 16
NW = NC * NS
N_PAD = 10240
E_PER_W = E // NW + 240
E_PAD = NW * E_PER_W
CHUNK = 128
CH_PER_W = E_PER_W // CHUNK
CH_GRP = 16
CH_C0 = 0
CH_C1 = 160
ROWS_PER_TILE = N_PAD // NS

_mesh = plsc.VectorSubcoreMesh(
    core_axis_name="c", subcore_axis_name="s", num_cores=NC, num_subcores=NS)
_sc_params = pltpu.CompilerParams(needs_layout_passes=False)


def _degree_body(src_hbm, dst_hbm, degs_hbm, degd_hbm, idx_s, idx_d, hist_s, hist_d):
    c = lax.axis_index("c")
    s = lax.axis_index("s")
    wid = c * NS + s
    ones = jnp.ones((L,), jnp.float32)

    def zero(i, _):
        hist_s[pl.ds(i * L, L)] = jnp.zeros((L,), jnp.float32)
        hist_d[pl.ds(i * L, L)] = jnp.zeros((L,), jnp.float32)
        return 0
    lax.fori_loop(0, N_PAD // L, zero, 0, unroll=4)

    pltpu.sync_copy(src_hbm.at[pl.ds(wid * E_PER_W, E_PER_W)], idx_s)
    pltpu.sync_copy(dst_hbm.at[pl.ds(wid * E_PER_W, E_PER_W)], idx_d)

    def hist(j, _):
        vs = idx_s[pl.ds(j * L, L)]
        plsc.addupdate_scatter(hist_s, [vs], ones)
        vd = idx_d[pl.ds(j * L, L)]
        plsc.addupdate_scatter(hist_d, [vd], ones)
        return 0
    lax.fori_loop(0, E_PER_W // L, hist, 0, unroll=4)

    pltpu.sync_copy(hist_s, degs_hbm.at[wid])
    pltpu.sync_copy(hist_d, degd_hbm.at[wid])


_degree_call = pl.kernel(
    _degree_body,
    out_type=[jax.ShapeDtypeStruct((NW, N_PAD), jnp.float32),
              jax.ShapeDtypeStruct((NW, N_PAD), jnp.float32)],
    mesh=_mesh,
    scratch_types=[pltpu.VMEM((E_PER_W,), jnp.int32),
                   pltpu.VMEM((E_PER_W,), jnp.int32),
                   pltpu.VMEM((N_PAD,), jnp.float32),
                   pltpu.VMEM((N_PAD,), jnp.float32)],
    compiler_params=_sc_params,
)


def _scatter_body(xs_hbm, src2_hbm, dst2_hbm, part_hbm,
                  idx_s, idx_d, buf0, buf1, zbuf, shared, sem0, sem1):
    c = lax.axis_index("c")
    s = lax.axis_index("s")

    for r in range(L):
        for l in range(D // L):
            zbuf[r, pl.ds(l * L, L)] = jnp.zeros((L,), jnp.float32)

    def zero(i, _):
        pltpu.sync_copy(zbuf, shared.at[pl.ds(s * ROWS_PER_TILE + i * L, L)])
        return 0
    lax.fori_loop(0, ROWS_PER_TILE // L, zero, 0)
    plsc.subcore_barrier()

    npair = CH_GRP // 2

    def run_groups(base_ch, ngrp):
        def group(g, _):
            gbase = base_ch + g * CH_GRP
            pltpu.sync_copy(src2_hbm.at[pl.ds(gbase, CH_GRP)], idx_s)
            pltpu.sync_copy(dst2_hbm.at[pl.ds(gbase, CH_GRP)], idx_d)
            pltpu.async_copy(xs_hbm.at[idx_s.at[0]], buf0, sem0)

            def edges(j, _):
                pltpu.async_copy(xs_hbm.at[idx_s.at[2 * j + 1]], buf1, sem1)
                pltpu.make_async_copy(xs_hbm.at[pl.ds(0, CHUNK)], buf0, sem0).wait()
                pltpu.sync_copy(buf0, shared.at[idx_d.at[2 * j]], add=True)

                @pl.when(j < npair - 1)
                def _():
                    pltpu.async_copy(xs_hbm.at[idx_s.at[2 * j + 2]], buf0, sem0)
                pltpu.make_async_copy(xs_hbm.at[pl.ds(0, CHUNK)], buf1, sem1).wait()
                pltpu.sync_copy(buf1, shared.at[idx_d.at[2 * j + 1]], add=True)
                return 0
            lax.fori_loop(0, npair, edges, 0)
            return 0
        lax.fori_loop(0, ngrp, group, 0)

    @pl.when(c == 0)
    def _():
        run_groups(s * CH_C0, CH_C0 // CH_GRP)

    @pl.when(c == 1)
    def _():
        run_groups(NS * CH_C0 + s * CH_C1, CH_C1 // CH_GRP)
    plsc.subcore_barrier()

    pltpu.sync_copy(shared.at[pl.ds(s * ROWS_PER_TILE, ROWS_PER_TILE)],
                    part_hbm.at[c].at[pl.ds(s * ROWS_PER_TILE, ROWS_PER_TILE)])


_scatter_call = pl.kernel(
    _scatter_body,
    out_type=jax.ShapeDtypeStruct((NC, N_PAD, D), jnp.float32),
    mesh=_mesh,
    scratch_types=[pltpu.VMEM((CH_GRP, CHUNK), jnp.int32),
                   pltpu.VMEM((CH_GRP, CHUNK), jnp.int32),
                   pltpu.VMEM((CHUNK, D), jnp.float32),
                   pltpu.VMEM((CHUNK, D), jnp.float32),
                   pltpu.VMEM((L, D), jnp.float32),
                   pltpu.VMEM_SHARED((N_PAD, D), jnp.float32),
                   pltpu.SemaphoreType.DMA,
                   pltpu.SemaphoreType.DMA],
    compiler_params=_sc_params,
)


def _norms_body(hs_ref, hd_ref, ns_ref, nd_ref):
    ds_ = jnp.sum(hs_ref[...], axis=0)
    dd = jnp.sum(hd_ref[...], axis=0)
    ns_ref[...] = jnp.where(ds_ > 0, lax.rsqrt(jnp.maximum(ds_, 1.0)), 0.0)
    nd_ref[...] = jnp.where(dd > 0, lax.rsqrt(jnp.maximum(dd, 1.0)), 0.0)


_norms_call = pl.pallas_call(
    _norms_body,
    out_shape=[jax.ShapeDtypeStruct((N_PAD,), jnp.float32),
               jax.ShapeDtypeStruct((N_PAD,), jnp.float32)],
)


def _pre_body(h_ref, w_ref, ns_ref, xs_ref):
    x = jnp.dot(h_ref[...], w_ref[...], preferred_element_type=jnp.float32)
    xs_ref[0:N, :] = x * ns_ref[0:N, :]
    xs_ref[N:, :] = jnp.zeros((N_PAD - N, D), jnp.float32)


_pre_call = pl.pallas_call(
    _pre_body,
    out_shape=jax.ShapeDtypeStruct((N_PAD, D), jnp.float32),
)


def _post(p_ref, nd_ref, sn_ref, b_ref, g_ref, be_ref, h_prev):
    agg = p_ref[0, 0:N, :] + p_ref[1, 0:N, :]
    x = agg * nd_ref[0:N, :] + b_ref[...]
    x = x * sn_ref[...]
    mean = jnp.mean(x, axis=0)
    var = jnp.mean((x - mean) ** 2, axis=0)
    x = (x - mean) * lax.rsqrt(var + 1e-5) * g_ref[...] + be_ref[...]
    return h_prev + jnp.maximum(x, 0.0)


def _mid_body(p_ref, nd_ref, sn_ref, b_ref, g_ref, be_ref, h0_ref, ns_ref,
              w1_ref, xs1_ref, h1_ref):
    h1 = _post(p_ref, nd_ref, sn_ref, b_ref, g_ref, be_ref, h0_ref[...])
    h1_ref[...] = h1
    x1 = jnp.dot(h1, w1_ref[...], preferred_element_type=jnp.float32)
    xs1_ref[0:N, :] = x1 * ns_ref[0:N, :]
    xs1_ref[N:, :] = jnp.zeros((N_PAD - N, D), jnp.float32)


_mid_call = pl.pallas_call(
    _mid_body,
    out_shape=[jax.ShapeDtypeStruct((N_PAD, D), jnp.float32),
               jax.ShapeDtypeStruct((N, D), jnp.float32)],
)


def _final_body(p_ref, nd_ref, sn_ref, b_ref, g_ref, be_ref, h1_ref, hg_ref):
    h2 = _post(p_ref, nd_ref, sn_ref, b_ref, g_ref, be_ref, h1_ref[...])
    hg_ref[...] = jnp.mean(h2, axis=0, keepdims=True)


_final_call = pl.pallas_call(
    _final_body,
    out_shape=jax.ShapeDtypeStruct((1, D), jnp.float32),
)


def kernel(nodes_feat, edge_index, edges_feat, nodes_num_norm_sqrt,
           edges_num_norm_sqrt, W0, b0, gamma0, beta0, W1, b1, gamma1, beta1):
    del edges_feat, edges_num_norm_sqrt
    src = edge_index[0]
    dst = edge_index[1]
    padv = jnp.full((E_PAD - E,), N, jnp.int32)
    src1 = jnp.concatenate([src, padv])
    dst1 = jnp.concatenate([dst, padv])
    src2 = src1.reshape(NW * CH_PER_W, CHUNK)
    dst2 = dst1.reshape(NW * CH_PER_W, CHUNK)

    degs, degd = _degree_call(src1, dst1)
    ns, nd = _norms_call(degs, degd)
    ns_col = ns.reshape(N_PAD, 1)
    nd_col = nd.reshape(N_PAD, 1)

    xs0 = _pre_call(nodes_feat, W0, ns_col)
    part0 = _scatter_call(xs0, src2, dst2)
    xs1, h1 = _mid_call(part0, nd_col, nodes_num_norm_sqrt, b0, gamma0, beta0,
                        nodes_feat, ns_col, W1)
    part1 = _scatter_call(xs1, src2, dst2)
    return _final_call(part1, nd_col, nodes_num_norm_sqrt, b1, gamma1, beta1, h1)


# speedup vs baseline: 3.6665x; 3.6665x over previous
import functools

import jax
import jax.numpy as jnp
from jax import lax
from jax.experimental import pallas as pl
from jax.experimental.pallas import tpu as pltpu
from jax.experimental.pallas import tpu_sc as plsc

N = 10000
E = 320000
D = 128
NC, NS, L = 2, 16, 16
NW = NC * NS
N_PAD = 10240
E_PER_W = E // NW + 240
E_PAD = NW * E_PER_W
CHUNK = 128
CH_PER_W = E_PER_W // CHUNK
CH_GRP = 16
CH_C0 = 80
CH_C1 = 80
ROWS_PER_TILE = N_PAD // NS

_mesh = plsc.VectorSubcoreMesh(
    core_axis_name="c", subcore_axis_name="s", num_cores=NC, num_subcores=NS)
_sc_params = pltpu.CompilerParams(needs_layout_passes=False)


def _degree_body(src_hbm, dst_hbm, degs_hbm, degd_hbm, idx_s, idx_d, hist_s, hist_d):
    c = lax.axis_index("c")
    s = lax.axis_index("s")
    wid = c * NS + s
    ones = jnp.ones((L,), jnp.float32)

    def zero(i, _):
        hist_s[pl.ds(i * L, L)] = jnp.zeros((L,), jnp.float32)
        hist_d[pl.ds(i * L, L)] = jnp.zeros((L,), jnp.float32)
        return 0
    lax.fori_loop(0, N_PAD // L, zero, 0, unroll=4)

    pltpu.sync_copy(src_hbm.at[pl.ds(wid * E_PER_W, E_PER_W)], idx_s)
    pltpu.sync_copy(dst_hbm.at[pl.ds(wid * E_PER_W, E_PER_W)], idx_d)

    def hist(j, _):
        vs = idx_s[pl.ds(j * L, L)]
        plsc.addupdate_scatter(hist_s, [vs], ones)
        vd = idx_d[pl.ds(j * L, L)]
        plsc.addupdate_scatter(hist_d, [vd], ones)
        return 0
    lax.fori_loop(0, E_PER_W // L, hist, 0, unroll=4)

    pltpu.sync_copy(hist_s, degs_hbm.at[wid])
    pltpu.sync_copy(hist_d, degd_hbm.at[wid])


_degree_call = pl.kernel(
    _degree_body,
    out_type=[jax.ShapeDtypeStruct((NW, N_PAD), jnp.float32),
              jax.ShapeDtypeStruct((NW, N_PAD), jnp.float32)],
    mesh=_mesh,
    scratch_types=[pltpu.VMEM((E_PER_W,), jnp.int32),
                   pltpu.VMEM((E_PER_W,), jnp.int32),
                   pltpu.VMEM((N_PAD,), jnp.float32),
                   pltpu.VMEM((N_PAD,), jnp.float32)],
    compiler_params=_sc_params,
)


def _scatter_body(xs_hbm, src2_hbm, dst2_hbm, part_hbm,
                  idx_s, idx_d, buf0, buf1, zbuf, shared, sem0, sem1):
    c = lax.axis_index("c")
    s = lax.axis_index("s")

    for r in range(L):
        for l in range(D // L):
            zbuf[r, pl.ds(l * L, L)] = jnp.zeros((L,), jnp.float32)

    def zero(i, _):
        pltpu.sync_copy(zbuf, shared.at[pl.ds(s * ROWS_PER_TILE + i * L, L)])
        return 0
    lax.fori_loop(0, ROWS_PER_TILE // L, zero, 0)
    plsc.subcore_barrier()

    npair = CH_GRP // 2

    def run_groups(base_ch, ngrp):
        def group(g, _):
            gbase = base_ch + g * CH_GRP
            pltpu.sync_copy(src2_hbm.at[pl.ds(gbase, CH_GRP)], idx_s)
            pltpu.sync_copy(dst2_hbm.at[pl.ds(gbase, CH_GRP)], idx_d)
            pltpu.async_copy(xs_hbm.at[idx_s.at[0]], buf0, sem0)

            def edges(j, _):
                pltpu.async_copy(xs_hbm.at[idx_s.at[2 * j + 1]], buf1, sem1)
                pltpu.make_async_copy(xs_hbm.at[pl.ds(0, CHUNK)], buf0, sem0).wait()
                pltpu.sync_copy(buf0, shared.at[idx_d.at[2 * j]], add=True)

                @pl.when(j < npair - 1)
                def _():
                    pltpu.async_copy(xs_hbm.at[idx_s.at[2 * j + 2]], buf0, sem0)
                pltpu.make_async_copy(xs_hbm.at[pl.ds(0, CHUNK)], buf1, sem1).wait()
                pltpu.sync_copy(buf1, shared.at[idx_d.at[2 * j + 1]], add=True)
                return 0
            lax.fori_loop(0, npair, edges, 0)
            return 0
        lax.fori_loop(0, ngrp, group, 0)

    @pl.when(c == 0)
    def _():
        run_groups(s * CH_C0, CH_C0 // CH_GRP)

    @pl.when(c == 1)
    def _():
        run_groups(NS * CH_C0 + s * CH_C1, CH_C1 // CH_GRP)
    plsc.subcore_barrier()

    pltpu.sync_copy(shared.at[pl.ds(s * ROWS_PER_TILE, ROWS_PER_TILE)],
                    part_hbm.at[c].at[pl.ds(s * ROWS_PER_TILE, ROWS_PER_TILE)])


_scatter_call = pl.kernel(
    _scatter_body,
    out_type=jax.ShapeDtypeStruct((NC, N_PAD, D), jnp.float32),
    mesh=_mesh,
    scratch_types=[pltpu.VMEM((CH_GRP, CHUNK), jnp.int32),
                   pltpu.VMEM((CH_GRP, CHUNK), jnp.int32),
                   pltpu.VMEM((CHUNK, D), jnp.float32),
                   pltpu.VMEM((CHUNK, D), jnp.float32),
                   pltpu.VMEM((L, D), jnp.float32),
                   pltpu.VMEM_SHARED((N_PAD, D), jnp.float32),
                   pltpu.SemaphoreType.DMA,
                   pltpu.SemaphoreType.DMA],
    compiler_params=_sc_params,
)


def _norms_body(hs_ref, hd_ref, ns_ref, nd_ref):
    ds_ = jnp.sum(hs_ref[...], axis=0)
    dd = jnp.sum(hd_ref[...], axis=0)
    ns_ref[...] = jnp.where(ds_ > 0, lax.rsqrt(jnp.maximum(ds_, 1.0)), 0.0)
    nd_ref[...] = jnp.where(dd > 0, lax.rsqrt(jnp.maximum(dd, 1.0)), 0.0)


_norms_call = pl.pallas_call(
    _norms_body,
    out_shape=[jax.ShapeDtypeStruct((N_PAD,), jnp.float32),
               jax.ShapeDtypeStruct((N_PAD,), jnp.float32)],
)


def _pre_body(h_ref, w_ref, ns_ref, xs_ref):
    x = jnp.dot(h_ref[...], w_ref[...], preferred_element_type=jnp.float32)
    xs_ref[0:N, :] = x * ns_ref[0:N, :]
    xs_ref[N:, :] = jnp.zeros((N_PAD - N, D), jnp.float32)


_pre_call = pl.pallas_call(
    _pre_body,
    out_shape=jax.ShapeDtypeStruct((N_PAD, D), jnp.float32),
)


def _post(p_ref, nd_ref, sn_ref, b_ref, g_ref, be_ref, h_prev):
    agg = p_ref[0, 0:N, :] + p_ref[1, 0:N, :]
    x = agg * nd_ref[0:N, :] + b_ref[...]
    x = x * sn_ref[...]
    mean = jnp.mean(x, axis=0)
    var = jnp.mean((x - mean) ** 2, axis=0)
    x = (x - mean) * lax.rsqrt(var + 1e-5) * g_ref[...] + be_ref[...]
    return h_prev + jnp.maximum(x, 0.0)


def _mid_body(p_ref, nd_ref, sn_ref, b_ref, g_ref, be_ref, h0_ref, ns_ref,
              w1_ref, xs1_ref, h1_ref):
    h1 = _post(p_ref, nd_ref, sn_ref, b_ref, g_ref, be_ref, h0_ref[...])
    h1_ref[...] = h1
    x1 = jnp.dot(h1, w1_ref[...], preferred_element_type=jnp.float32)
    xs1_ref[0:N, :] = x1 * ns_ref[0:N, :]
    xs1_ref[N:, :] = jnp.zeros((N_PAD - N, D), jnp.float32)


_mid_call = pl.pallas_call(
    _mid_body,
    out_shape=[jax.ShapeDtypeStruct((N_PAD, D), jnp.float32),
               jax.ShapeDtypeStruct((N, D), jnp.float32)],
)


def _final_body(p_ref, nd_ref, sn_ref, b_ref, g_ref, be_ref, h1_ref, hg_ref):
    h2 = _post(p_ref, nd_ref, sn_ref, b_ref, g_ref, be_ref, h1_ref[...])
    hg_ref[...] = jnp.mean(h2, axis=0, keepdims=True)


_final_call = pl.pallas_call(
    _final_body,
    out_shape=jax.ShapeDtypeStruct((1, D), jnp.float32),
)


def kernel(nodes_feat, edge_index, edges_feat, nodes_num_norm_sqrt,
           edges_num_norm_sqrt, W0, b0, gamma0, beta0, W1, b1, gamma1, beta1):
    del edges_feat, edges_num_norm_sqrt
    src = edge_index[0]
    dst = edge_index[1]
    n_pad_w = E_PER_W - E // NW
    padv = (N + jnp.arange(NW * n_pad_w, dtype=jnp.int32) % (N_PAD - N)
            ).reshape(NW, n_pad_w)
    src1 = jnp.concatenate([src.reshape(NW, E // NW), padv], axis=1).reshape(-1)
    dst1 = jnp.concatenate([dst.reshape(NW, E // NW), padv], axis=1).reshape(-1)
    src2 = src1.reshape(NW * CH_PER_W, CHUNK)
    dst2 = dst1.reshape(NW * CH_PER_W, CHUNK)

    degs, degd = _degree_call(src1, dst1)
    ns, nd = _norms_call(degs, degd)
    ns_col = ns.reshape(N_PAD, 1)
    nd_col = nd.reshape(N_PAD, 1)

    xs0 = _pre_call(nodes_feat, W0, ns_col)
    part0 = _scatter_call(xs0, src2, dst2)
    xs1, h1 = _mid_call(part0, nd_col, nodes_num_norm_sqrt, b0, gamma0, beta0,
                        nodes_feat, ns_col, W1)
    part1 = _scatter_call(xs1, src2, dst2)
    return _final_call(part1, nd_col, nodes_num_norm_sqrt, b1, gamma1, beta1, h1)


# speedup vs baseline: 3.8284x; 1.0441x over previous
import jax
import jax.numpy as jnp
from jax import lax
from jax.experimental import pallas as pl
from jax.experimental.pallas import tpu as pltpu
from jax.experimental.pallas import tpu_sc as plsc

N = 10000
E = 320000
D = 128
NC, NS, L = 2, 16, 16
NW = NC * NS
N_PAD = 10240
E_PER_W = 10240
E_PAD = NW * E_PER_W
CHUNK = 128
CH_PER_W = E_PER_W // CHUNK
CH_GRP = 16
N_GRP = CH_PER_W // CH_GRP
ROWS_PER_TILE = N_PAD // NS

_mesh = plsc.VectorSubcoreMesh(
    core_axis_name="c", subcore_axis_name="s", num_cores=NC, num_subcores=NS)
_sc_params = pltpu.CompilerParams(needs_layout_passes=False)


def _degree_body(src_hbm, dst_hbm, degs_hbm, degd_hbm, idx_s, idx_d, hist_s, hist_d):
    c = lax.axis_index("c")
    s = lax.axis_index("s")
    wid = c * NS + s
    ones = jnp.ones((L,), jnp.float32)

    def zero(i, _):
        hist_s[pl.ds(i * L, L)] = jnp.zeros((L,), jnp.float32)
        hist_d[pl.ds(i * L, L)] = jnp.zeros((L,), jnp.float32)
        return 0
    lax.fori_loop(0, N_PAD // L, zero, 0, unroll=4)

    pltpu.sync_copy(src_hbm.at[pl.ds(wid * CH_PER_W, CH_PER_W)], idx_s)
    pltpu.sync_copy(dst_hbm.at[pl.ds(wid * CH_PER_W, CH_PER_W)], idx_d)

    def hist(j, _):
        for h in range(CHUNK // L):
            vs = idx_s[j, pl.ds(h * L, L)]
            plsc.addupdate_scatter(hist_s, [vs], ones)
            vd = idx_d[j, pl.ds(h * L, L)]
            plsc.addupdate_scatter(hist_d, [vd], ones)
        return 0
    lax.fori_loop(0, CH_PER_W, hist, 0)

    pltpu.sync_copy(hist_s, degs_hbm.at[wid])
    pltpu.sync_copy(hist_d, degd_hbm.at[wid])


_degree_call = pl.kernel(
    _degree_body,
    out_type=[jax.ShapeDtypeStruct((NW, N_PAD), jnp.float32),
              jax.ShapeDtypeStruct((NW, N_PAD), jnp.float32)],
    mesh=_mesh,
    scratch_types=[pltpu.VMEM((CH_PER_W, CHUNK), jnp.int32),
                   pltpu.VMEM((CH_PER_W, CHUNK), jnp.int32),
                   pltpu.VMEM((N_PAD,), jnp.float32),
                   pltpu.VMEM((N_PAD,), jnp.float32)],
    compiler_params=_sc_params,
)


def _scatter_body(xs_hbm, src2_hbm, dst2_hbm, part_hbm,
                  isa0, ida0, isa1, ida1, buf0, buf1, zbuf, shared,
                  sem0, sem1, isem0, isem1, zsem):
    c = lax.axis_index("c")
    s = lax.axis_index("s")
    wid = c * NS + s
    base_ch = wid * CH_PER_W

    for r in range(L):
        for l in range(D // L):
            zbuf[r, pl.ds(l * L, L)] = jnp.zeros((L,), jnp.float32)

    def zfire(i, _):
        pltpu.async_copy(zbuf, shared.at[pl.ds(s * ROWS_PER_TILE + i * L, L)], zsem)
        return 0
    lax.fori_loop(0, ROWS_PER_TILE // L, zfire, 0)

    def load_idx(g, bs, bd, sem):
        pltpu.async_copy(src2_hbm.at[pl.ds(base_ch + g * CH_GRP, CH_GRP)], bs, sem)
        pltpu.async_copy(dst2_hbm.at[pl.ds(base_ch + g * CH_GRP, CH_GRP)], bd, sem)

    def wait_idx(bs, bd, sem):
        pltpu.make_async_copy(src2_hbm.at[pl.ds(0, CH_GRP)], bs, sem).wait()
        pltpu.make_async_copy(src2_hbm.at[pl.ds(0, CH_GRP)], bd, sem).wait()

    load_idx(0, isa0, ida0, isem0)

    def zdrain(i, _):
        pltpu.make_async_copy(xs_hbm.at[pl.ds(0, L)], zbuf, zsem).wait()
        return 0
    lax.fori_loop(0, ROWS_PER_TILE // L, zdrain, 0)
    plsc.subcore_barrier()

    def proc(bs, bd):
        pltpu.async_copy(xs_hbm.at[bs.at[0]], buf0, sem0)

        def edges(j, _):
            pltpu.async_copy(xs_hbm.at[bs.at[2 * j + 1]], buf1, sem1)
            pltpu.make_async_copy(xs_hbm.at[pl.ds(0, CHUNK)], buf0, sem0).wait()
            pltpu.sync_copy(buf0, shared.at[bd.at[2 * j]], add=True)

            @pl.when(j < CH_GRP // 2 - 1)
            def _():
                pltpu.async_copy(xs_hbm.at[bs.at[2 * j + 2]], buf0, sem0)
            pltpu.make_async_copy(xs_hbm.at[pl.ds(0, CHUNK)], buf1, sem1).wait()
            pltpu.sync_copy(buf1, shared.at[bd.at[2 * j + 1]], add=True)
            return 0
        lax.fori_loop(0, CH_GRP // 2, edges, 0)

    def gpair(gp, _):
        load_idx(2 * gp + 1, isa1, ida1, isem1)
        wait_idx(isa0, ida0, isem0)
        proc(isa0, ida0)
        load_idx(2 * gp + 2, isa0, ida0, isem0)
        wait_idx(isa1, ida1, isem1)
        proc(isa1, ida1)
        return 0
    lax.fori_loop(0, (N_GRP - 1) // 2, gpair, 0)
    wait_idx(isa0, ida0, isem0)
    proc(isa0, ida0)
    plsc.subcore_barrier()

    pltpu.sync_copy(shared.at[pl.ds(s * ROWS_PER_TILE, ROWS_PER_TILE)],
                    part_hbm.at[c].at[pl.ds(s * ROWS_PER_TILE, ROWS_PER_TILE)])


_scatter_call = pl.kernel(
    _scatter_body,
    out_type=jax.ShapeDtypeStruct((NC, N_PAD, D), jnp.float32),
    mesh=_mesh,
    scratch_types=[pltpu.VMEM((CH_GRP, CHUNK), jnp.int32),
                   pltpu.VMEM((CH_GRP, CHUNK), jnp.int32),
                   pltpu.VMEM((CH_GRP, CHUNK), jnp.int32),
                   pltpu.VMEM((CH_GRP, CHUNK), jnp.int32),
                   pltpu.VMEM((CHUNK, D), jnp.float32),
                   pltpu.VMEM((CHUNK, D), jnp.float32),
                   pltpu.VMEM((L, D), jnp.float32),
                   pltpu.VMEM_SHARED((N_PAD, D), jnp.float32),
                   pltpu.SemaphoreType.DMA,
                   pltpu.SemaphoreType.DMA,
                   pltpu.SemaphoreType.DMA,
                   pltpu.SemaphoreType.DMA,
                   pltpu.SemaphoreType.DMA],
    compiler_params=_sc_params,
)


def _norms_body(hs_ref, hd_ref, ns_ref, nd_ref):
    ds_ = jnp.sum(hs_ref[...], axis=0)
    dd = jnp.sum(hd_ref[...], axis=0)
    ns_ref[...] = jnp.where(ds_ > 0, lax.rsqrt(jnp.maximum(ds_, 1.0)), 0.0)
    nd_ref[...] = jnp.where(dd > 0, lax.rsqrt(jnp.maximum(dd, 1.0)), 0.0)


_norms_call = pl.pallas_call(
    _norms_body,
    out_shape=[jax.ShapeDtypeStruct((N_PAD,), jnp.float32),
               jax.ShapeDtypeStruct((N_PAD,), jnp.float32)],
)


def _matmul_body(h_ref, w_ref, x_ref):
    x_ref[...] = jnp.dot(h_ref[...], w_ref[...],
                         preferred_element_type=jnp.float32)


_matmul_call = pl.pallas_call(
    _matmul_body,
    out_shape=jax.ShapeDtypeStruct((N, D), jnp.float32),
)


def _scale_body(x_ref, ns_ref, xs_ref):
    xs_ref[0:N, :] = x_ref[...] * ns_ref[0:N, :]
    xs_ref[N:, :] = jnp.zeros((N_PAD - N, D), jnp.float32)


_scale_call = pl.pallas_call(
    _scale_body,
    out_shape=jax.ShapeDtypeStruct((N_PAD, D), jnp.float32),
)


def _post(p_ref, nd_ref, sn_ref, b_ref, g_ref, be_ref, h_prev):
    agg = p_ref[0, 0:N, :] + p_ref[1, 0:N, :]
    x = agg * nd_ref[0:N, :] + b_ref[...]
    x = x * sn_ref[...]
    mean = jnp.mean(x, axis=0)
    var = jnp.mean((x - mean) ** 2, axis=0)
    x = (x - mean) * lax.rsqrt(var + 1e-5) * g_ref[...] + be_ref[...]
    return h_prev + jnp.maximum(x, 0.0)


def _mid_body(p_ref, nd_ref, sn_ref, b_ref, g_ref, be_ref, h0_ref, ns_ref,
              w1_ref, xs1_ref, h1_ref):
    h1 = _post(p_ref, nd_ref, sn_ref, b_ref, g_ref, be_ref, h0_ref[...])
    h1_ref[...] = h1
    x1 = jnp.dot(h1, w1_ref[...], preferred_element_type=jnp.float32)
    xs1_ref[0:N, :] = x1 * ns_ref[0:N, :]
    xs1_ref[N:, :] = jnp.zeros((N_PAD - N, D), jnp.float32)


_mid_call = pl.pallas_call(
    _mid_body,
    out_shape=[jax.ShapeDtypeStruct((N_PAD, D), jnp.float32),
               jax.ShapeDtypeStruct((N, D), jnp.float32)],
)


def _final_body(p_ref, nd_ref, sn_ref, b_ref, g_ref, be_ref, h1_ref, hg_ref):
    h2 = _post(p_ref, nd_ref, sn_ref, b_ref, g_ref, be_ref, h1_ref[...])
    hg_ref[...] = jnp.mean(h2, axis=0, keepdims=True)


_final_call = pl.pallas_call(
    _final_body,
    out_shape=jax.ShapeDtypeStruct((1, D), jnp.float32),
)


def kernel(nodes_feat, edge_index, edges_feat, nodes_num_norm_sqrt,
           edges_num_norm_sqrt, W0, b0, gamma0, beta0, W1, b1, gamma1, beta1):
    del edges_feat, edges_num_norm_sqrt
    src = edge_index[0]
    dst = edge_index[1]
    n_pad_w = E_PER_W - E // NW
    padv = (N + jnp.arange(NW * n_pad_w, dtype=jnp.int32) % (N_PAD - N)
            ).reshape(NW, n_pad_w)
    src2 = jnp.concatenate([src.reshape(NW, E // NW), padv],
                           axis=1).reshape(NW * CH_PER_W, CHUNK)
    dst2 = jnp.concatenate([dst.reshape(NW, E // NW), padv],
                           axis=1).reshape(NW * CH_PER_W, CHUNK)

    degs, degd = _degree_call(src2, dst2)
    x0 = _matmul_call(nodes_feat, W0)
    ns, nd = _norms_call(degs, degd)
    ns_col = ns.reshape(N_PAD, 1)
    nd_col = nd.reshape(N_PAD, 1)

    xs0 = _scale_call(x0, ns_col)
    part0 = _scatter_call(xs0, src2, dst2)
    xs1, h1 = _mid_call(part0, nd_col, nodes_num_norm_sqrt, b0, gamma0, beta0,
                        nodes_feat, ns_col, W1)
    part1 = _scatter_call(xs1, src2, dst2)
    return _final_call(part1, nd_col, nodes_num_norm_sqrt, b1, gamma1, beta1, h1)


# speedup vs baseline: 3.8707x; 1.0111x over previous
import jax
import jax.numpy as jnp
from jax import lax
from jax.experimental import pallas as pl
from jax.experimental.pallas import tpu as pltpu
from jax.experimental.pallas import tpu_sc as plsc

N = 10000
E = 320000
D = 128
NC, NS, L = 2, 16, 16
NW = NC * NS
N_PAD = 10240
E_PER_W = 10240
E_PAD = NW * E_PER_W
CHUNK = 128
CH_PER_W = E_PER_W // CHUNK
CH_GRP = 16
N_GRP = CH_PER_W // CH_GRP
ROWS_PER_TILE = N_PAD // NS

_mesh = plsc.VectorSubcoreMesh(
    core_axis_name="c", subcore_axis_name="s", num_cores=NC, num_subcores=NS)
_sc_params = pltpu.CompilerParams(needs_layout_passes=False)


E_REAL_W = E // NW


def _degree_body(src_hbm, dst_hbm, degs_hbm, degd_hbm, idx_s, idx_d, hist_s, hist_d):
    c = lax.axis_index("c")
    s = lax.axis_index("s")
    wid = c * NS + s
    ones = jnp.ones((L,), jnp.float32)

    def zero(i, _):
        hist_s[pl.ds(i * L, L)] = jnp.zeros((L,), jnp.float32)
        hist_d[pl.ds(i * L, L)] = jnp.zeros((L,), jnp.float32)
        return 0
    lax.fori_loop(0, N_PAD // L, zero, 0, unroll=4)

    pltpu.sync_copy(src_hbm.at[pl.ds(wid * E_REAL_W, E_REAL_W)], idx_s)
    pltpu.sync_copy(dst_hbm.at[pl.ds(wid * E_REAL_W, E_REAL_W)], idx_d)

    def hist(j, _):
        vs = idx_s[pl.ds(j * L, L)]
        plsc.addupdate_scatter(hist_s, [vs], ones)
        vd = idx_d[pl.ds(j * L, L)]
        plsc.addupdate_scatter(hist_d, [vd], ones)
        return 0
    lax.fori_loop(0, E_REAL_W // L, hist, 0, unroll=4)

    pltpu.sync_copy(hist_s, degs_hbm.at[wid])
    pltpu.sync_copy(hist_d, degd_hbm.at[wid])


_degree_call = pl.kernel(
    _degree_body,
    out_type=[jax.ShapeDtypeStruct((NW, N_PAD), jnp.float32),
              jax.ShapeDtypeStruct((NW, N_PAD), jnp.float32)],
    mesh=_mesh,
    scratch_types=[pltpu.VMEM((E_REAL_W,), jnp.int32),
                   pltpu.VMEM((E_REAL_W,), jnp.int32),
                   pltpu.VMEM((N_PAD,), jnp.float32),
                   pltpu.VMEM((N_PAD,), jnp.float32)],
    compiler_params=_sc_params,
)


def _scatter_body(xs_hbm, src2_hbm, dst2_hbm, part_hbm,
                  isa0, ida0, isa1, ida1, buf0, buf1, zbuf, shared,
                  sem0, sem1, isem0, isem1, zsem):
    c = lax.axis_index("c")
    s = lax.axis_index("s")
    wid = c * NS + s
    base_ch = wid * CH_PER_W

    for r in range(L):
        for l in range(D // L):
            zbuf[r, pl.ds(l * L, L)] = jnp.zeros((L,), jnp.float32)

    def zfire(i, _):
        pltpu.async_copy(zbuf, shared.at[pl.ds(s * ROWS_PER_TILE + i * L, L)], zsem)
        return 0
    lax.fori_loop(0, ROWS_PER_TILE // L, zfire, 0)

    def load_idx(g, bs, bd, sem):
        pltpu.async_copy(src2_hbm.at[pl.ds(base_ch + g * CH_GRP, CH_GRP)], bs, sem)
        pltpu.async_copy(dst2_hbm.at[pl.ds(base_ch + g * CH_GRP, CH_GRP)], bd, sem)

    def wait_idx(bs, bd, sem):
        pltpu.make_async_copy(src2_hbm.at[pl.ds(0, CH_GRP)], bs, sem).wait()
        pltpu.make_async_copy(src2_hbm.at[pl.ds(0, CH_GRP)], bd, sem).wait()

    load_idx(0, isa0, ida0, isem0)

    def zdrain(i, _):
        pltpu.make_async_copy(xs_hbm.at[pl.ds(0, L)], zbuf, zsem).wait()
        return 0
    lax.fori_loop(0, ROWS_PER_TILE // L, zdrain, 0)
    plsc.subcore_barrier()

    def proc(bs, bd):
        pltpu.async_copy(xs_hbm.at[bs.at[0]], buf0, sem0)

        def edges(j, _):
            pltpu.async_copy(xs_hbm.at[bs.at[2 * j + 1]], buf1, sem1)
            pltpu.make_async_copy(xs_hbm.at[pl.ds(0, CHUNK)], buf0, sem0).wait()
            pltpu.sync_copy(buf0, shared.at[bd.at[2 * j]], add=True)

            @pl.when(j < CH_GRP // 2 - 1)
            def _():
                pltpu.async_copy(xs_hbm.at[bs.at[2 * j + 2]], buf0, sem0)
            pltpu.make_async_copy(xs_hbm.at[pl.ds(0, CHUNK)], buf1, sem1).wait()
            pltpu.sync_copy(buf1, shared.at[bd.at[2 * j + 1]], add=True)
            return 0
        lax.fori_loop(0, CH_GRP // 2, edges, 0)

    def gpair(gp, _):
        load_idx(2 * gp + 1, isa1, ida1, isem1)
        wait_idx(isa0, ida0, isem0)
        proc(isa0, ida0)
        load_idx(2 * gp + 2, isa0, ida0, isem0)
        wait_idx(isa1, ida1, isem1)
        proc(isa1, ida1)
        return 0
    lax.fori_loop(0, (N_GRP - 1) // 2, gpair, 0)
    wait_idx(isa0, ida0, isem0)
    proc(isa0, ida0)
    plsc.subcore_barrier()

    pltpu.sync_copy(shared.at[pl.ds(s * ROWS_PER_TILE, ROWS_PER_TILE)],
                    part_hbm.at[c].at[pl.ds(s * ROWS_PER_TILE, ROWS_PER_TILE)])


_scatter_call = pl.kernel(
    _scatter_body,
    out_type=jax.ShapeDtypeStruct((NC, N_PAD, D), jnp.float32),
    mesh=_mesh,
    scratch_types=[pltpu.VMEM((CH_GRP, CHUNK), jnp.int32),
                   pltpu.VMEM((CH_GRP, CHUNK), jnp.int32),
                   pltpu.VMEM((CH_GRP, CHUNK), jnp.int32),
                   pltpu.VMEM((CH_GRP, CHUNK), jnp.int32),
                   pltpu.VMEM((CHUNK, D), jnp.float32),
                   pltpu.VMEM((CHUNK, D), jnp.float32),
                   pltpu.VMEM((L, D), jnp.float32),
                   pltpu.VMEM_SHARED((N_PAD, D), jnp.float32),
                   pltpu.SemaphoreType.DMA,
                   pltpu.SemaphoreType.DMA,
                   pltpu.SemaphoreType.DMA,
                   pltpu.SemaphoreType.DMA,
                   pltpu.SemaphoreType.DMA],
    compiler_params=_sc_params,
)


def _norms_body(hs_ref, hd_ref, ns_ref, nd_ref):
    ds_ = jnp.sum(hs_ref[...], axis=0)
    dd = jnp.sum(hd_ref[...], axis=0)
    ns_ref[...] = jnp.where(ds_ > 0, lax.rsqrt(jnp.maximum(ds_, 1.0)), 0.0)
    nd_ref[...] = jnp.where(dd > 0, lax.rsqrt(jnp.maximum(dd, 1.0)), 0.0)


_norms_call = pl.pallas_call(
    _norms_body,
    out_shape=[jax.ShapeDtypeStruct((N_PAD,), jnp.float32),
               jax.ShapeDtypeStruct((N_PAD,), jnp.float32)],
)


def _matmul_body(h_ref, w_ref, x_ref):
    x_ref[...] = jnp.dot(h_ref[...], w_ref[...],
                         preferred_element_type=jnp.float32)


_matmul_call = pl.pallas_call(
    _matmul_body,
    out_shape=jax.ShapeDtypeStruct((N, D), jnp.float32),
)


def _scale_body(x_ref, ns_ref, xs_ref):
    xs_ref[0:N, :] = x_ref[...] * ns_ref[0:N, :]
    xs_ref[N:, :] = jnp.zeros((N_PAD - N, D), jnp.float32)


_scale_call = pl.pallas_call(
    _scale_body,
    out_shape=jax.ShapeDtypeStruct((N_PAD, D), jnp.float32),
)


def _post(p_ref, nd_ref, sn_ref, b_ref, g_ref, be_ref, h_prev):
    agg = p_ref[0, 0:N, :] + p_ref[1, 0:N, :]
    x = agg * nd_ref[0:N, :] + b_ref[...]
    x = x * sn_ref[...]
    mean = jnp.mean(x, axis=0)
    var = jnp.mean((x - mean) ** 2, axis=0)
    x = (x - mean) * lax.rsqrt(var + 1e-5) * g_ref[...] + be_ref[...]
    return h_prev + jnp.maximum(x, 0.0)


def _mid_body(p_ref, nd_ref, sn_ref, b_ref, g_ref, be_ref, h0_ref, ns_ref,
              w1_ref, xs1_ref, h1_ref):
    h1 = _post(p_ref, nd_ref, sn_ref, b_ref, g_ref, be_ref, h0_ref[...])
    h1_ref[...] = h1
    x1 = jnp.dot(h1, w1_ref[...], preferred_element_type=jnp.float32)
    xs1_ref[0:N, :] = x1 * ns_ref[0:N, :]
    xs1_ref[N:, :] = jnp.zeros((N_PAD - N, D), jnp.float32)


_mid_call = pl.pallas_call(
    _mid_body,
    out_shape=[jax.ShapeDtypeStruct((N_PAD, D), jnp.float32),
               jax.ShapeDtypeStruct((N, D), jnp.float32)],
)


def _final_body(p_ref, nd_ref, sn_ref, b_ref, g_ref, be_ref, h1_ref, hg_ref):
    h2 = _post(p_ref, nd_ref, sn_ref, b_ref, g_ref, be_ref, h1_ref[...])
    hg_ref[...] = jnp.mean(h2, axis=0, keepdims=True)


_final_call = pl.pallas_call(
    _final_body,
    out_shape=jax.ShapeDtypeStruct((1, D), jnp.float32),
)


def kernel(nodes_feat, edge_index, edges_feat, nodes_num_norm_sqrt,
           edges_num_norm_sqrt, W0, b0, gamma0, beta0, W1, b1, gamma1, beta1):
    del edges_feat, edges_num_norm_sqrt
    src = edge_index[0]
    dst = edge_index[1]
    n_pad_w = E_PER_W - E // NW
    padv = (N + jnp.arange(NW * n_pad_w, dtype=jnp.int32) % (N_PAD - N)
            ).reshape(NW, n_pad_w)
    src2 = jnp.concatenate([src.reshape(NW, E // NW), padv],
                           axis=1).reshape(NW * CH_PER_W, CHUNK)
    dst2 = jnp.concatenate([dst.reshape(NW, E // NW), padv],
                           axis=1).reshape(NW * CH_PER_W, CHUNK)

    degs, degd = _degree_call(src, dst)
    x0 = _matmul_call(nodes_feat, W0)
    ns, nd = _norms_call(degs, degd)
    ns_col = ns.reshape(N_PAD, 1)
    nd_col = nd.reshape(N_PAD, 1)

    xs0 = _scale_call(x0, ns_col)
    part0 = _scatter_call(xs0, src2, dst2)
    xs1, h1 = _mid_call(part0, nd_col, nodes_num_norm_sqrt, b0, gamma0, beta0,
                        nodes_feat, ns_col, W1)
    part1 = _scatter_call(xs1, src2, dst2)
    return _final_call(part1, nd_col, nodes_num_norm_sqrt, b1, gamma1, beta1, h1)


# speedup vs baseline: 3.9685x; 1.0253x over previous
import jax
import jax.numpy as jnp
from jax import lax
from jax.experimental import pallas as pl
from jax.experimental.pallas import tpu as pltpu
from jax.experimental.pallas import tpu_sc as plsc

N = 10000
E = 320000
D = 128
NC, NS, L = 2, 16, 16
NW = NC * NS
N_PAD = 10240
E_PER_W = 10240
E_PAD = NW * E_PER_W
CHUNK = 128
CH_PER_W = E_PER_W // CHUNK
CH_GRP = 16
N_GRP = CH_PER_W // CH_GRP
ROWS_PER_TILE = N_PAD // NS

_mesh = plsc.VectorSubcoreMesh(
    core_axis_name="c", subcore_axis_name="s", num_cores=NC, num_subcores=NS)
_sc_params = pltpu.CompilerParams(needs_layout_passes=False)


E_REAL_W = E // NW


def _degree_body(src_hbm, dst_hbm, degs_hbm, degd_hbm, idx_s, idx_d, hist_s, hist_d):
    c = lax.axis_index("c")
    s = lax.axis_index("s")
    wid = c * NS + s
    ones = jnp.ones((L,), jnp.float32)

    def zero(i, _):
        hist_s[pl.ds(i * L, L)] = jnp.zeros((L,), jnp.float32)
        hist_d[pl.ds(i * L, L)] = jnp.zeros((L,), jnp.float32)
        return 0
    lax.fori_loop(0, N_PAD // L, zero, 0, unroll=4)

    pltpu.sync_copy(src_hbm.at[pl.ds(wid * E_REAL_W, E_REAL_W)], idx_s)
    pltpu.sync_copy(dst_hbm.at[pl.ds(wid * E_REAL_W, E_REAL_W)], idx_d)

    def hist(j, _):
        vs = idx_s[pl.ds(j * L, L)]
        plsc.addupdate_scatter(hist_s, [vs], ones)
        vd = idx_d[pl.ds(j * L, L)]
        plsc.addupdate_scatter(hist_d, [vd], ones)
        return 0
    lax.fori_loop(0, E_REAL_W // L, hist, 0, unroll=4)

    pltpu.sync_copy(hist_s, degs_hbm.at[wid])
    pltpu.sync_copy(hist_d, degd_hbm.at[wid])


_degree_call = pl.kernel(
    _degree_body,
    out_type=[jax.ShapeDtypeStruct((NW, N_PAD), jnp.float32),
              jax.ShapeDtypeStruct((NW, N_PAD), jnp.float32)],
    mesh=_mesh,
    scratch_types=[pltpu.VMEM((E_REAL_W,), jnp.int32),
                   pltpu.VMEM((E_REAL_W,), jnp.int32),
                   pltpu.VMEM((N_PAD,), jnp.float32),
                   pltpu.VMEM((N_PAD,), jnp.float32)],
    compiler_params=_sc_params,
)


def _scatter_body(xs_hbm, src2_hbm, dst2_hbm, part_hbm,
                  isa0, ida0, isa1, ida1, buf0, buf1, zbuf, shared,
                  sem0, sem1, isem0, isem1, zsem):
    c = lax.axis_index("c")
    s = lax.axis_index("s")
    wid = c * NS + s
    base_ch = wid * CH_PER_W

    for r in range(L):
        for l in range(D // L):
            zbuf[r, pl.ds(l * L, L)] = jnp.zeros((L,), jnp.float32)

    def zfire(i, _):
        pltpu.async_copy(zbuf, shared.at[pl.ds(s * ROWS_PER_TILE + i * L, L)], zsem)
        return 0
    lax.fori_loop(0, ROWS_PER_TILE // L, zfire, 0)

    def load_idx(g, bs, bd, sem):
        pltpu.async_copy(src2_hbm.at[pl.ds(base_ch + g * CH_GRP, CH_GRP)], bs, sem)
        pltpu.async_copy(dst2_hbm.at[pl.ds(base_ch + g * CH_GRP, CH_GRP)], bd, sem)

    def wait_idx(bs, bd, sem):
        pltpu.make_async_copy(src2_hbm.at[pl.ds(0, CH_GRP)], bs, sem).wait()
        pltpu.make_async_copy(src2_hbm.at[pl.ds(0, CH_GRP)], bd, sem).wait()

    load_idx(0, isa0, ida0, isem0)

    def zdrain(i, _):
        pltpu.make_async_copy(xs_hbm.at[pl.ds(0, L)], zbuf, zsem).wait()
        return 0
    lax.fori_loop(0, ROWS_PER_TILE // L, zdrain, 0)
    plsc.subcore_barrier()

    def proc(bs, bd):
        pltpu.async_copy(xs_hbm.at[bs.at[0]], buf0, sem0)

        def edges(j, _):
            pltpu.async_copy(xs_hbm.at[bs.at[2 * j + 1]], buf1, sem1)
            pltpu.make_async_copy(xs_hbm.at[pl.ds(0, CHUNK)], buf0, sem0).wait()
            pltpu.sync_copy(buf0, shared.at[bd.at[2 * j]], add=True)

            @pl.when(j < CH_GRP // 2 - 1)
            def _():
                pltpu.async_copy(xs_hbm.at[bs.at[2 * j + 2]], buf0, sem0)
            pltpu.make_async_copy(xs_hbm.at[pl.ds(0, CHUNK)], buf1, sem1).wait()
            pltpu.sync_copy(buf1, shared.at[bd.at[2 * j + 1]], add=True)
            return 0
        lax.fori_loop(0, CH_GRP // 2, edges, 0)

    def gpair(gp, _):
        load_idx(2 * gp + 1, isa1, ida1, isem1)
        wait_idx(isa0, ida0, isem0)
        proc(isa0, ida0)
        load_idx(2 * gp + 2, isa0, ida0, isem0)
        wait_idx(isa1, ida1, isem1)
        proc(isa1, ida1)
        return 0
    lax.fori_loop(0, (N_GRP - 1) // 2, gpair, 0)
    wait_idx(isa0, ida0, isem0)
    proc(isa0, ida0)
    plsc.subcore_barrier()

    pltpu.sync_copy(shared.at[pl.ds(s * ROWS_PER_TILE, ROWS_PER_TILE)],
                    part_hbm.at[c].at[pl.ds(s * ROWS_PER_TILE, ROWS_PER_TILE)])


_scatter_call = pl.kernel(
    _scatter_body,
    out_type=jax.ShapeDtypeStruct((NC, N_PAD, D), jnp.float32),
    mesh=_mesh,
    scratch_types=[pltpu.VMEM((CH_GRP, CHUNK), jnp.int32),
                   pltpu.VMEM((CH_GRP, CHUNK), jnp.int32),
                   pltpu.VMEM((CH_GRP, CHUNK), jnp.int32),
                   pltpu.VMEM((CH_GRP, CHUNK), jnp.int32),
                   pltpu.VMEM((CHUNK, D), jnp.float32),
                   pltpu.VMEM((CHUNK, D), jnp.float32),
                   pltpu.VMEM((L, D), jnp.float32),
                   pltpu.VMEM_SHARED((N_PAD, D), jnp.float32),
                   pltpu.SemaphoreType.DMA,
                   pltpu.SemaphoreType.DMA,
                   pltpu.SemaphoreType.DMA,
                   pltpu.SemaphoreType.DMA,
                   pltpu.SemaphoreType.DMA],
    compiler_params=_sc_params,
)


def _norms_body(hs_ref, hd_ref, ns_ref, nd_ref):
    ds_ = jnp.sum(hs_ref[...], axis=0)
    dd = jnp.sum(hd_ref[...], axis=0)
    ns_ref[...] = jnp.where(ds_ > 0, lax.rsqrt(jnp.maximum(ds_, 1.0)), 0.0)
    nd_ref[...] = jnp.where(dd > 0, lax.rsqrt(jnp.maximum(dd, 1.0)), 0.0)


_norms_call = pl.pallas_call(
    _norms_body,
    out_shape=[jax.ShapeDtypeStruct((N_PAD,), jnp.float32),
               jax.ShapeDtypeStruct((N_PAD,), jnp.float32)],
)


def _matmul_body(h_ref, w_ref, x_ref):
    x_ref[...] = jnp.dot(h_ref[...], w_ref[...],
                         preferred_element_type=jnp.float32)


_matmul_call = pl.pallas_call(
    _matmul_body,
    out_shape=jax.ShapeDtypeStruct((N, D), jnp.float32),
)


def _scale_body(x_ref, hs_ref, xs_ref):
    ds_ = jnp.sum(hs_ref[...], axis=0)
    ns = jnp.where(ds_ > 0, lax.rsqrt(jnp.maximum(ds_, 1.0)), 0.0)
    ns_col = ns[0:N].reshape(N, 1)
    xs_ref[0:N, :] = x_ref[...] * ns_col
    xs_ref[N:, :] = jnp.zeros((N_PAD - N, D), jnp.float32)


_scale_call = pl.pallas_call(
    _scale_body,
    out_shape=jax.ShapeDtypeStruct((N_PAD, D), jnp.float32),
)


def _post(p_ref, nd_ref, sn_ref, b_ref, g_ref, be_ref, h_prev):
    agg = p_ref[0, 0:N, :] + p_ref[1, 0:N, :]
    x = agg * nd_ref[0:N, :] + b_ref[...]
    x = x * sn_ref[...]
    mean = jnp.mean(x, axis=0)
    var = jnp.mean((x - mean) ** 2, axis=0)
    x = (x - mean) * lax.rsqrt(var + 1e-5) * g_ref[...] + be_ref[...]
    return h_prev + jnp.maximum(x, 0.0)


def _mid_body(p_ref, nd_ref, sn_ref, b_ref, g_ref, be_ref, h0_ref, ns_ref,
              w1_ref, xs1_ref, h1_ref):
    h1 = _post(p_ref, nd_ref, sn_ref, b_ref, g_ref, be_ref, h0_ref[...])
    h1_ref[...] = h1
    x1 = jnp.dot(h1, w1_ref[...], preferred_element_type=jnp.float32)
    xs1_ref[0:N, :] = x1 * ns_ref[0:N, :]
    xs1_ref[N:, :] = jnp.zeros((N_PAD - N, D), jnp.float32)


_mid_call = pl.pallas_call(
    _mid_body,
    out_shape=[jax.ShapeDtypeStruct((N_PAD, D), jnp.float32),
               jax.ShapeDtypeStruct((N, D), jnp.float32)],
)


def _final_body(p_ref, nd_ref, sn_ref, b_ref, g_ref, be_ref, h1_ref, hg_ref):
    h2 = _post(p_ref, nd_ref, sn_ref, b_ref, g_ref, be_ref, h1_ref[...])
    hg_ref[...] = jnp.mean(h2, axis=0, keepdims=True)


_final_call = pl.pallas_call(
    _final_body,
    out_shape=jax.ShapeDtypeStruct((1, D), jnp.float32),
)


def kernel(nodes_feat, edge_index, edges_feat, nodes_num_norm_sqrt,
           edges_num_norm_sqrt, W0, b0, gamma0, beta0, W1, b1, gamma1, beta1):
    del edges_feat, edges_num_norm_sqrt
    src = edge_index[0]
    dst = edge_index[1]
    n_pad_w = E_PER_W - E // NW
    padv = (N + jnp.arange(NW * n_pad_w, dtype=jnp.int32) % (N_PAD - N)
            ).reshape(NW, n_pad_w)
    src2 = jnp.concatenate([src.reshape(NW, E // NW), padv],
                           axis=1).reshape(NW * CH_PER_W, CHUNK)
    dst2 = jnp.concatenate([dst.reshape(NW, E // NW), padv],
                           axis=1).reshape(NW * CH_PER_W, CHUNK)

    degs, degd = _degree_call(src, dst)
    x0 = _matmul_call(nodes_feat, W0)
    ns, nd = _norms_call(degs, degd)
    ns_col = ns.reshape(N_PAD, 1)
    nd_col = nd.reshape(N_PAD, 1)

    xs0 = _scale_call(x0, degs)
    part0 = _scatter_call(xs0, src2, dst2)
    xs1, h1 = _mid_call(part0, nd_col, nodes_num_norm_sqrt, b0, gamma0, beta0,
                        nodes_feat, ns_col, W1)
    part1 = _scatter_call(xs1, src2, dst2)
    return _final_call(part1, nd_col, nodes_num_norm_sqrt, b1, gamma1, beta1, h1)
